# baseline (speedup 1.0000x reference)
import functools

import jax
import jax.numpy as jnp
from jax import lax
from jax.experimental import pallas as pl
from jax.experimental.pallas import tpu as pltpu
from jax.experimental.pallas import tpu_sc as plsc

VOCAB = 100000
HIDDEN = 128
EPS = 1e-12

NUM_CORES = 2
NUM_SUBCORES = 16
NW = NUM_CORES * NUM_SUBCORES
C = 128
LANES = 16
VPH = HIDDEN // LANES


def _permute(v, idx2d):
    return lax.gather(
        v, idx2d,
        lax.GatherDimensionNumbers(
            offset_dims=(), collapsed_slice_dims=(0,), start_index_map=(0,)),
        (1,), mode=lax.GatherScatterMode.PROMISE_IN_BOUNDS)


def _rsqrt_vec(x):
    i = lax.bitcast_convert_type(x, jnp.int32)
    i = jnp.int32(0x5F3759DF) - (i >> 1)
    y = lax.bitcast_convert_type(i, jnp.float32)
    half = x * jnp.float32(0.5)
    for _ in range(3):
        y = y * (jnp.float32(1.5) - half * y * y)
    return y


def _sc_body(ids_w, ids_p, ids_t, wtab, ptab, ttab, gamma, beta, out,
             idxw_v, idxp_v, idxt_v, w_rows, p_rows, t_rows, g_v, b_v, sem,
             *, n_tokens):
    wid = lax.axis_index("s") * NUM_CORES + lax.axis_index("c")
    per_w = n_tokens // NW
    chunks = per_w // C
    base0 = wid * per_w

    pltpu.sync_copy(gamma, g_v)
    pltpu.sync_copy(beta, b_v)

    inv_h = jnp.float32(1.0 / HIDDEN)

    def chunk_body(g, _):
        base = base0 + g * C
        pltpu.sync_copy(ids_w.at[pl.ds(base, C)], idxw_v)
        pltpu.sync_copy(ids_p.at[pl.ds(base, C)], idxp_v)
        pltpu.sync_copy(ids_t.at[pl.ds(base, C)], idxt_v)
        cw = pltpu.async_copy(wtab.at[idxw_v], w_rows, sem)
        cp = pltpu.async_copy(ptab.at[idxp_v], p_rows, sem)
        ct = pltpu.async_copy(ttab.at[idxt_v], t_rows, sem)
        cw.wait()
        cp.wait()
        ct.wait()

        lane = lax.iota(jnp.int32, LANES)
        perms = [(lane ^ (1 << k)).reshape(LANES, 1) for k in range(4)]

        def tok_body(t, _):
            accs = []
            vsum = jnp.zeros((LANES,), jnp.float32)
            vsq = jnp.zeros((LANES,), jnp.float32)
            for j in range(VPH):
                sl = pl.ds(j * LANES, LANES)
                a = w_rows[t, sl] + p_rows[t, sl] + t_rows[t, sl]
                accs.append(a)
                vsum = vsum + a
                vsq = vsq + a * a
            for p in perms:
                vsum = vsum + _permute(vsum, p)
                vsq = vsq + _permute(vsq, p)
            mv = vsum * inv_h
            var = vsq * inv_h - mv * mv
            rstd = _rsqrt_vec(var + jnp.float32(EPS))
            for j in range(VPH):
                sl = pl.ds(j * LANES, LANES)
                w_rows[t, sl] = (accs[j] - mv) * rstd * g_v[sl] + b_v[sl]
            return _

        lax.fori_loop(0, C, tok_body, None)
        pltpu.sync_copy(w_rows, out.at[pl.ds(base, C)])
        return _

    lax.fori_loop(0, chunks, chunk_body, None)


def kernel(input_ids, token_type_ids, position_ids, word_emb, pos_emb,
           type_emb, ln_gamma, ln_beta):
    B, L = input_ids.shape
    n = B * L
    ids_w = input_ids.reshape(n).astype(jnp.int32)
    ids_t = token_type_ids.reshape(n).astype(jnp.int32)
    ids_p = position_ids.reshape(n).astype(jnp.int32)

    mesh = plsc.VectorSubcoreMesh(
        core_axis_name="c", subcore_axis_name="s",
        num_cores=NUM_CORES, num_subcores=NUM_SUBCORES)

    run = pl.kernel(
        functools.partial(_sc_body, n_tokens=n),
        out_type=jax.ShapeDtypeStruct((n, HIDDEN), jnp.float32),
        mesh=mesh,
        scratch_types=[
            pltpu.VMEM((C,), jnp.int32),
            pltpu.VMEM((C,), jnp.int32),
            pltpu.VMEM((C,), jnp.int32),
            pltpu.VMEM((C, HIDDEN), jnp.float32),
            pltpu.VMEM((C, HIDDEN), jnp.float32),
            pltpu.VMEM((C, HIDDEN), jnp.float32),
            pltpu.VMEM((HIDDEN,), jnp.float32),
            pltpu.VMEM((HIDDEN,), jnp.float32),
            pltpu.SemaphoreType.DMA,
        ],
    )
    out = run(ids_w, ids_p, ids_t, word_emb, pos_emb, type_emb,
              ln_gamma, ln_beta)
    return out.reshape(B, L, HIDDEN)


# speedup vs baseline: 1.0013x; 1.0013x over previous
import functools

import jax
import jax.numpy as jnp
from jax import lax
from jax.experimental import pallas as pl
from jax.experimental.pallas import tpu as pltpu
from jax.experimental.pallas import tpu_sc as plsc

VOCAB = 100000
HIDDEN = 128
EPS = 1e-12

NUM_CORES = 2
NUM_SUBCORES = 16
NW = NUM_CORES * NUM_SUBCORES
C = 128
UNROLL = 4
LANES = 16
VPH = HIDDEN // LANES


def _permute(v, idx2d):
    return lax.gather(
        v, idx2d,
        lax.GatherDimensionNumbers(
            offset_dims=(), collapsed_slice_dims=(0,), start_index_map=(0,)),
        (1,), mode=lax.GatherScatterMode.PROMISE_IN_BOUNDS)


def _rsqrt_vec(x):
    i = lax.bitcast_convert_type(x, jnp.int32)
    i = jnp.int32(0x5F3759DF) - (i >> 1)
    y = lax.bitcast_convert_type(i, jnp.float32)
    half = x * jnp.float32(0.5)
    for _ in range(2):
        y = y * (jnp.float32(1.5) - half * y * y)
    return y


def _sc_body(ids_w, ids_p, ids_t, wtab, ptab, ttab, gamma, beta, out,
             idxw_v, idxp_v, idxt_v, w_rows, p_rows, t_rows, g_v, b_v, sem,
             *, n_tokens):
    wid = lax.axis_index("s") * NUM_CORES + lax.axis_index("c")
    per_w = n_tokens // NW
    chunks = per_w // C
    base0 = wid * per_w

    pltpu.sync_copy(gamma, g_v)
    pltpu.sync_copy(beta, b_v)

    inv_h = jnp.float32(1.0 / HIDDEN)

    def chunk_body(g, _):
        base = base0 + g * C
        pltpu.sync_copy(ids_w.at[pl.ds(base, C)], idxw_v)
        pltpu.sync_copy(ids_p.at[pl.ds(base, C)], idxp_v)
        pltpu.sync_copy(ids_t.at[pl.ds(base, C)], idxt_v)
        cw = pltpu.async_copy(wtab.at[idxw_v], w_rows, sem)
        cp = pltpu.async_copy(ptab.at[idxp_v], p_rows, sem)
        ct = pltpu.async_copy(ttab.at[idxt_v], t_rows, sem)
        cw.wait()
        cp.wait()
        ct.wait()

        lane = lax.iota(jnp.int32, LANES)
        perms = [(lane ^ (1 << k)).reshape(LANES, 1) for k in range(4)]

        def tok_body(ti, _):
            for u in range(UNROLL):
                t = ti * UNROLL + u
                accs = []
                vsum = jnp.zeros((LANES,), jnp.float32)
                vsq = jnp.zeros((LANES,), jnp.float32)
                for j in range(VPH):
                    sl = pl.ds(j * LANES, LANES)
                    a = w_rows[t, sl] + p_rows[t, sl] + t_rows[t, sl]
                    accs.append(a)
                    vsum = vsum + a
                    vsq = vsq + a * a
                for p in perms:
                    vsum = vsum + _permute(vsum, p)
                    vsq = vsq + _permute(vsq, p)
                mv = vsum * inv_h
                var = vsq * inv_h - mv * mv
                rstd = _rsqrt_vec(var + jnp.float32(EPS))
                for j in range(VPH):
                    sl = pl.ds(j * LANES, LANES)
                    w_rows[t, sl] = (accs[j] - mv) * rstd * g_v[sl] + b_v[sl]
            return _

        lax.fori_loop(0, C // UNROLL, tok_body, None)
        pltpu.sync_copy(w_rows, out.at[pl.ds(base, C)])
        return _

    lax.fori_loop(0, chunks, chunk_body, None)


def kernel(input_ids, token_type_ids, position_ids, word_emb, pos_emb,
           type_emb, ln_gamma, ln_beta):
    B, L = input_ids.shape
    n = B * L
    ids_w = input_ids.reshape(n).astype(jnp.int32)
    ids_t = token_type_ids.reshape(n).astype(jnp.int32)
    ids_p = position_ids.reshape(n).astype(jnp.int32)

    mesh = plsc.VectorSubcoreMesh(
        core_axis_name="c", subcore_axis_name="s",
        num_cores=NUM_CORES, num_subcores=NUM_SUBCORES)

    run = pl.kernel(
        functools.partial(_sc_body, n_tokens=n),
        out_type=jax.ShapeDtypeStruct((n, HIDDEN), jnp.float32),
        mesh=mesh,
        scratch_types=[
            pltpu.VMEM((C,), jnp.int32),
            pltpu.VMEM((C,), jnp.int32),
            pltpu.VMEM((C,), jnp.int32),
            pltpu.VMEM((C, HIDDEN), jnp.float32),
            pltpu.VMEM((C, HIDDEN), jnp.float32),
            pltpu.VMEM((C, HIDDEN), jnp.float32),
            pltpu.VMEM((HIDDEN,), jnp.float32),
            pltpu.VMEM((HIDDEN,), jnp.float32),
            pltpu.SemaphoreType.DMA,
        ],
    )
    out = run(ids_w, ids_p, ids_t, word_emb, pos_emb, type_emb,
              ln_gamma, ln_beta)
    return out.reshape(B, L, HIDDEN)


# speedup vs baseline: 11.1055x; 11.0913x over previous
import functools

import jax
import jax.numpy as jnp
from jax import lax
from jax.experimental import pallas as pl
from jax.experimental.pallas import tpu as pltpu
from jax.experimental.pallas import tpu_sc as plsc

HIDDEN = 128
EPS = 1e-12

NUM_CORES = 2
NUM_SUBCORES = 16
NW = NUM_CORES * NUM_SUBCORES
C = 128
LANES = 16
VPH = HIDDEN // LANES
TGROUP = 16


def _permute(v, idx2d):
    return lax.gather(
        v, idx2d,
        lax.GatherDimensionNumbers(
            offset_dims=(), collapsed_slice_dims=(0,), start_index_map=(0,)),
        (1,), mode=lax.GatherScatterMode.PROMISE_IN_BOUNDS)


def _rsqrt_vec(x):
    i = lax.bitcast_convert_type(x, jnp.int32)
    i = jnp.int32(0x5F3759DF) - (i >> 1)
    y = lax.bitcast_convert_type(i, jnp.float32)
    half = x * jnp.float32(0.5)
    for _ in range(2):
        y = y * (jnp.float32(1.5) - half * y * y)
    return y


def _sc_body(ids_w3, ids_p3, ids_t3, wtab, ptab, ttab, gamma, beta, out,
             idsw_v, idsp_v, idst_v, wb, pb, ob, g_v, b_v, t_v,
             semg, semwb, *, n_tokens):
    wid = lax.axis_index("s") * NUM_CORES + lax.axis_index("c")
    per_w = n_tokens // NW
    ch = per_w // C
    base0 = wid * per_w

    pltpu.sync_copy(gamma, g_v)
    pltpu.sync_copy(beta, b_v)
    pltpu.sync_copy(ttab, t_v)
    pltpu.sync_copy(ids_w3.at[wid], idsw_v)
    pltpu.sync_copy(ids_p3.at[wid], idsp_v)
    pltpu.sync_copy(ids_t3.at[wid], idst_v)

    inv_h = jnp.float32(1.0 / HIDDEN)
    lane = lax.iota(jnp.int32, LANES)
    perms = [(lane ^ (1 << k)).reshape(LANES, 1) for k in range(4)]
    splat_idx = [jnp.full((LANES, 1), u, jnp.int32) for u in range(TGROUP)]
    t0s = [t_v[pl.ds(j * LANES, LANES)] for j in range(VPH)]
    tds = [t_v[pl.ds(HIDDEN + j * LANES, LANES)] - t0s[j] for j in range(VPH)]

    def fire(g, par):
        pltpu.async_copy(wtab.at[idsw_v.at[g]], wb.at[par], semg.at[par])
        pltpu.async_copy(ptab.at[idsp_v.at[g]], pb.at[par], semg.at[par])

    def drain(g, par):
        pltpu.make_async_copy(wtab.at[idsw_v.at[g]], wb.at[par],
                              semg.at[par]).wait()
        pltpu.make_async_copy(ptab.at[idsp_v.at[g]], pb.at[par],
                              semg.at[par]).wait()

    def out_slice(g):
        return out.at[pl.ds(base0 + g * C, C)]

    def compute(g, par):
        def tok16(i16, _):
            tb = i16 * TGROUP
            tv16 = idst_v[g, pl.ds(tb, TGROUP)]
            for u in range(TGROUP):
                t = tb + u
                tidf = _permute(tv16, splat_idx[u]).astype(jnp.float32)
                accs = []
                vsum = jnp.zeros((LANES,), jnp.float32)
                vsq = jnp.zeros((LANES,), jnp.float32)
                for j in range(VPH):
                    sl = pl.ds(j * LANES, LANES)
                    a = (wb[par, t, sl] + pb[par, t, sl]
                         + (t0s[j] + tidf * tds[j]))
                    accs.append(a)
                    vsum = vsum + a
                    vsq = vsq + a * a
                for p in perms:
                    vsum = vsum + _permute(vsum, p)
                    vsq = vsq + _permute(vsq, p)
                mv = vsum * inv_h
                var = vsq * inv_h - mv * mv
                rstd = _rsqrt_vec(var + jnp.float32(EPS))
                for j in range(VPH):
                    sl = pl.ds(j * LANES, LANES)
                    ob[par, t, sl] = (accs[j] - mv) * rstd * g_v[sl] + b_v[sl]
            return _

        lax.fori_loop(0, C // TGROUP, tok16, None)

    fire(0, 0)

    def pipe(i, _):
        for par in (0, 1):
            g = i * 2 + par

            @pl.when(g + 1 < ch)
            def _(g=g, par=par):
                fire(g + 1, (par + 1) % 2)

            drain(g, par)

            @pl.when(g >= 2)
            def _(g=g, par=par):
                pltpu.make_async_copy(ob.at[par], out_slice(g - 2),
                                      semwb.at[par]).wait()

            compute(g, par)
            pltpu.async_copy(ob.at[par], out_slice(g), semwb.at[par])
        return _

    lax.fori_loop(0, ch // 2, pipe, None)

    for par, g in ((0, ch - 2), (1, ch - 1)):
        pltpu.make_async_copy(ob.at[par], out_slice(g), semwb.at[par]).wait()


def kernel(input_ids, token_type_ids, position_ids, word_emb, pos_emb,
           type_emb, ln_gamma, ln_beta):
    B, L = input_ids.shape
    n = B * L
    ch = n // NW // C
    ids_w = input_ids.reshape(NW, ch, C).astype(jnp.int32)
    ids_t = token_type_ids.reshape(NW, ch, C).astype(jnp.int32)
    ids_p = position_ids.reshape(NW, ch, C).astype(jnp.int32)

    mesh = plsc.VectorSubcoreMesh(
        core_axis_name="c", subcore_axis_name="s",
        num_cores=NUM_CORES, num_subcores=NUM_SUBCORES)

    run = pl.kernel(
        functools.partial(_sc_body, n_tokens=n),
        out_type=jax.ShapeDtypeStruct((n, HIDDEN), jnp.float32),
        mesh=mesh,
        scratch_types=[
            pltpu.VMEM((ch, C), jnp.int32),
            pltpu.VMEM((ch, C), jnp.int32),
            pltpu.VMEM((ch, C), jnp.int32),
            pltpu.VMEM((2, C, HIDDEN), jnp.float32),
            pltpu.VMEM((2, C, HIDDEN), jnp.float32),
            pltpu.VMEM((2, C, HIDDEN), jnp.float32),
            pltpu.VMEM((HIDDEN,), jnp.float32),
            pltpu.VMEM((HIDDEN,), jnp.float32),
            pltpu.VMEM((2 * HIDDEN,), jnp.float32),
            pltpu.SemaphoreType.DMA((2,)),
            pltpu.SemaphoreType.DMA((2,)),
        ],
    )
    out = run(ids_w, ids_p, ids_t, word_emb, pos_emb,
              type_emb.reshape(-1), ln_gamma, ln_beta)
    return out.reshape(B, L, HIDDEN)
